# Initial kernel scaffold; baseline (speedup 1.0000x reference)
import jax
import jax.numpy as jnp
from jax.experimental import pallas as pl


def kernel(x):
    raise NotImplementedError("write your pallas kernel here")


# speedup vs baseline: 1.0316x; 1.0316x over previous
import functools

import jax
import jax.numpy as jnp
from jax import lax
from jax.experimental import pallas as pl
from jax.experimental.pallas import tpu as pltpu
from jax.experimental.pallas import tpu_sc as plsc

L = 16384
V = 1000
NC, NS, LANES = 2, 16, 16
NW = NC * NS
RPW = L // NW
CH = 128
NCHUNK = RPW // CH
VPC = CH // LANES


_COL_OFFS = [o * LANES for o in range(V // LANES)] + [V - LANES]


def _body(x_hbm, out_hbm, idx_v, buf2d):
    wid = lax.axis_index("s") * NC + lax.axis_index("c")
    base = wid * RPW
    pltpu.sync_copy(x_hbm.at[pl.ds(base, RPW)], idx_v)

    zeros16 = jnp.zeros((LANES,), jnp.float32)
    ones16 = jnp.full((LANES,), 1.0, jnp.float32)

    def _zero(r, carry):
        for off in _COL_OFFS:
            buf2d[r, pl.ds(off, LANES)] = zeros16
        return carry

    lax.fori_loop(0, CH, _zero, 0)

    def _idx(c, j):
        iv = idx_v[pl.ds(c * CH + j * LANES, LANES)]
        rows = lax.iota(jnp.int32, LANES) + j * LANES
        return rows, iv

    for c in range(NCHUNK):
        for j in range(VPC):
            rows, iv = _idx(c, j)
            plsc.store_scatter(buf2d, [rows, iv], ones16)
        pltpu.sync_copy(buf2d, out_hbm.at[pl.ds(base + c * CH, CH)])
        for j in range(VPC):
            rows, iv = _idx(c, j)
            plsc.store_scatter(buf2d, [rows, iv], zeros16)


@jax.jit
def _one_hot_sc(xf):
    kfn = pl.kernel(
        _body,
        out_type=jax.ShapeDtypeStruct((L, V), jnp.float32),
        mesh=plsc.VectorSubcoreMesh(core_axis_name="c", subcore_axis_name="s"),
        scratch_types=[
            pltpu.VMEM((RPW,), jnp.int32),
            pltpu.VMEM((CH, V), jnp.float32),
        ],
        compiler_params=pltpu.CompilerParams(
            use_tc_tiling_on_sc=False, needs_layout_passes=False
        ),
    )
    return kfn(xf)


def kernel(x):
    return _one_hot_sc(x.reshape(L))


# speedup vs baseline: 3.7793x; 3.6636x over previous
import jax
import jax.numpy as jnp
from jax import lax
from jax.experimental import pallas as pl
from jax.experimental.pallas import tpu as pltpu
from jax.experimental.pallas import tpu_sc as plsc

L = 16384
V = 1000
NC, NS, LANES = 2, 16, 16
NW = NC * NS
CPW = L // NW
CCH = 128
NCHUNK = CPW // CCH
VPC = CCH // LANES


def _body(x_hbm, out_hbm, idx_v, buf):
    wid = lax.axis_index("s") * NC + lax.axis_index("c")
    base = wid * CPW
    pltpu.sync_copy(x_hbm.at[pl.ds(base, CPW)], idx_v)

    zeros16 = jnp.zeros((LANES,), jnp.float32)
    ones16 = jnp.full((LANES,), 1.0, jnp.float32)

    def _zero(r, carry):
        for off in range(0, CCH, LANES):
            buf[r, pl.ds(off, LANES)] = zeros16
        return carry

    lax.fori_loop(0, V, _zero, 0)

    def _idx(c, k):
        vv = idx_v[pl.ds(c * CCH + k * LANES, LANES)]
        cols = lax.iota(jnp.int32, LANES) + k * LANES
        return vv, cols

    for c in range(NCHUNK):
        for k in range(VPC):
            vv, cols = _idx(c, k)
            plsc.store_scatter(buf, [vv, cols], ones16)
        pltpu.sync_copy(buf, out_hbm.at[:, pl.ds(base + c * CCH, CCH)])
        for k in range(VPC):
            vv, cols = _idx(c, k)
            plsc.store_scatter(buf, [vv, cols], zeros16)


@jax.jit
def _one_hot_sc(xf):
    kfn = pl.kernel(
        _body,
        out_type=jax.ShapeDtypeStruct((V, L), jnp.float32),
        mesh=plsc.VectorSubcoreMesh(core_axis_name="c", subcore_axis_name="s"),
        scratch_types=[
            pltpu.VMEM((CPW,), jnp.int32),
            pltpu.VMEM((V, CCH), jnp.float32),
        ],
        compiler_params=pltpu.CompilerParams(
            use_tc_tiling_on_sc=True, needs_layout_passes=False
        ),
    )
    return kfn(xf)


def kernel(x):
    return _one_hot_sc(x.reshape(L)).T


# speedup vs baseline: 3.8826x; 1.0273x over previous
import jax
import jax.numpy as jnp
from jax import lax
from jax.experimental import pallas as pl
from jax.experimental.pallas import tpu as pltpu
from jax.experimental.pallas import tpu_sc as plsc

L = 16384
V = 1000
VA = 504
VB = V - VA
NC, NS, LANES = 2, 16, 16
NW = NC * NS
CPW = L // NW
CCH = 128
NCHUNK = CPW // CCH
VPC = CCH // LANES


def _body(x_hbm, out_hbm, idx_v, buf_a, buf_b, sem_a, sem_b):
    wid = lax.axis_index("s") * NC + lax.axis_index("c")
    base = wid * CPW
    pltpu.sync_copy(x_hbm.at[pl.ds(base, CPW)], idx_v)

    zeros16 = jnp.zeros((LANES,), jnp.float32)
    ones16 = jnp.full((LANES,), 1.0, jnp.float32)

    def _zero_rows(buf):
        def body(r, carry):
            for off in range(0, CCH, LANES):
                buf[r, pl.ds(off, LANES)] = zeros16
            return carry
        return body

    def _idx(c, k):
        vv = idx_v[pl.ds(c * CCH + k * LANES, LANES)]
        cols = lax.iota(jnp.int32, LANES) + k * LANES
        return vv, cols

    def _scatter_a(c, val):
        for k in range(VPC):
            vv, cols = _idx(c, k)
            plsc.store_scatter(buf_a, [vv, cols], val, mask=vv < VA)

    def _scatter_b(c, val):
        for k in range(VPC):
            vv, cols = _idx(c, k)
            plsc.store_scatter(buf_b, [vv - VA, cols], val, mask=vv >= VA)

    def _dma_a(c):
        return pltpu.async_copy(
            buf_a, out_hbm.at[pl.ds(0, VA), pl.ds(base + c * CCH, CCH)], sem_a
        )

    def _dma_b(c):
        return pltpu.async_copy(
            buf_b, out_hbm.at[pl.ds(VA, VB), pl.ds(base + c * CCH, CCH)], sem_b
        )

    lax.fori_loop(0, VA, _zero_rows(buf_a), 0)
    _scatter_a(0, ones16)
    cp_a = _dma_a(0)
    lax.fori_loop(0, VB, _zero_rows(buf_b), 0)
    _scatter_b(0, ones16)
    cp_b = _dma_b(0)
    for c in range(1, NCHUNK):
        cp_a.wait()
        _scatter_a(c - 1, zeros16)
        _scatter_a(c, ones16)
        cp_a = _dma_a(c)
        cp_b.wait()
        _scatter_b(c - 1, zeros16)
        _scatter_b(c, ones16)
        cp_b = _dma_b(c)
    cp_a.wait()
    cp_b.wait()


@jax.jit
def _one_hot_sc(xf):
    kfn = pl.kernel(
        _body,
        out_type=jax.ShapeDtypeStruct((V, L), jnp.float32),
        mesh=plsc.VectorSubcoreMesh(core_axis_name="c", subcore_axis_name="s"),
        scratch_types=[
            pltpu.VMEM((CPW,), jnp.int32),
            pltpu.VMEM((VA, CCH), jnp.float32),
            pltpu.VMEM((VB, CCH), jnp.float32),
            pltpu.SemaphoreType.DMA,
            pltpu.SemaphoreType.DMA,
        ],
        compiler_params=pltpu.CompilerParams(
            use_tc_tiling_on_sc=True, needs_layout_passes=False
        ),
    )
    return kfn(xf)


def kernel(x):
    return _one_hot_sc(x.reshape(L)).T


# speedup vs baseline: 4.0320x; 1.0385x over previous
import jax
import jax.numpy as jnp
from jax import lax
from jax.experimental import pallas as pl
from jax.experimental.pallas import tpu as pltpu
from jax.experimental.pallas import tpu_sc as plsc

L = 16384
V = 1000
VA = 504
VB = V - VA
NC, NS, LANES = 2, 16, 16
NW = NC * NS
CPW = L // NW
CCH = 128
NCHUNK = CPW // CCH
VPC = CCH // LANES

SEGS_A = [(0, 128), (128, 128), (256, 128), (384, 120)]
SEGS_B = [(0, 128), (128, 128), (256, 128), (384, 112)]


def _body(x_hbm, out_hbm, idx_v, buf_a, buf_b,
          sem_i, sem_a, sem_b, sems_ra, sems_rb):
    wid = lax.axis_index("s") * NC + lax.axis_index("c")
    base = wid * CPW
    cp_i = pltpu.async_copy(x_hbm.at[pl.ds(base, CPW)], idx_v, sem_i)

    zeros16 = jnp.zeros((LANES,), jnp.float32)
    ones16 = jnp.full((LANES,), 1.0, jnp.float32)

    def _zero_rows(buf, lo):
        def body(r, carry):
            for off in range(0, CCH, LANES):
                buf[lo + r, pl.ds(off, LANES)] = zeros16
            return carry
        return body

    def _idx(c, k):
        vv = idx_v[pl.ds(c * CCH + k * LANES, LANES)]
        cols = lax.iota(jnp.int32, LANES) + k * LANES
        return vv, cols

    def _scatter_a(c, val):
        for k in range(VPC):
            vv, cols = _idx(c, k)
            plsc.store_scatter(buf_a, [vv, cols], val, mask=vv < VA)

    def _scatter_b(c, val):
        for k in range(VPC):
            vv, cols = _idx(c, k)
            plsc.store_scatter(buf_b, [vv - VA, cols], val, mask=vv >= VA)

    def _scatter_seg(buf, vbase, lo, n, val):
        glo, ghi = vbase + lo, vbase + lo + n
        for k in range(VPC):
            vv, cols = _idx(0, k)
            m = (vv >= glo) & (vv < ghi)
            plsc.store_scatter(buf, [vv - vbase, cols], val, mask=m)

    def _dma_a(c):
        return pltpu.async_copy(
            buf_a, out_hbm.at[pl.ds(0, VA), pl.ds(base + c * CCH, CCH)], sem_a
        )

    def _dma_b(c):
        return pltpu.async_copy(
            buf_b, out_hbm.at[pl.ds(VA, VB), pl.ds(base + c * CCH, CCH)], sem_b
        )

    ramp = []
    first = True
    for buf, vbase, segs, sems in (
        (buf_a, 0, SEGS_A, sems_ra),
        (buf_b, VA, SEGS_B, sems_rb),
    ):
        for s, (lo, n) in enumerate(segs):
            lax.fori_loop(0, n, _zero_rows(buf, lo), 0)
            if first:
                cp_i.wait()
                first = False
            _scatter_seg(buf, vbase, lo, n, ones16)
            ramp.append(pltpu.async_copy(
                buf.at[pl.ds(lo, n)],
                out_hbm.at[pl.ds(vbase + lo, n), pl.ds(base, CCH)],
                sems[s],
            ))

    cp_a = cp_b = None
    for c in range(1, NCHUNK):
        if c == 1:
            for cp in ramp[:len(SEGS_A)]:
                cp.wait()
        else:
            cp_a.wait()
        _scatter_a(c - 1, zeros16)
        _scatter_a(c, ones16)
        cp_a = _dma_a(c)
        if c == 1:
            for cp in ramp[len(SEGS_A):]:
                cp.wait()
        else:
            cp_b.wait()
        _scatter_b(c - 1, zeros16)
        _scatter_b(c, ones16)
        cp_b = _dma_b(c)
    cp_a.wait()
    cp_b.wait()


@jax.jit
def _one_hot_sc(xf):
    kfn = pl.kernel(
        _body,
        out_type=jax.ShapeDtypeStruct((V, L), jnp.float32),
        mesh=plsc.VectorSubcoreMesh(core_axis_name="c", subcore_axis_name="s"),
        scratch_types=[
            pltpu.VMEM((CPW,), jnp.int32),
            pltpu.VMEM((VA, CCH), jnp.float32),
            pltpu.VMEM((VB, CCH), jnp.float32),
            pltpu.SemaphoreType.DMA,
            pltpu.SemaphoreType.DMA,
            pltpu.SemaphoreType.DMA,
            [pltpu.SemaphoreType.DMA] * len(SEGS_A),
            [pltpu.SemaphoreType.DMA] * len(SEGS_B),
        ],
        compiler_params=pltpu.CompilerParams(
            use_tc_tiling_on_sc=True, needs_layout_passes=False
        ),
    )
    return kfn(xf)


def kernel(x):
    return _one_hot_sc(x.reshape(L)).T
